# baseline (speedup 1.0000x reference)
import functools

import jax
import jax.numpy as jnp
from jax import lax
from jax.experimental import pallas as pl
from jax.experimental.pallas import tpu as pltpu
from jax.experimental.pallas import tpu_sc as plsc

_BETA = 0.25
_BM = 256
_BK = 512
_BOUNDS = (2736, 5472, 8192)


def _argmin_body(zf_ref, emb_ref, idx_ref, acc_ref, esq_ref):
    i = pl.program_id(0)
    kk = emb_ref.shape[0]

    @pl.when(i == 0)
    def _():
        esq_ref[:] = jnp.sum(emb_ref[:] ** 2, axis=1)
        acc_ref[0, 0] = 0.0

    zblk = zf_ref[0]
    zsq = jnp.sum(zblk ** 2, axis=1)

    accv = jnp.full((_BM,), jnp.inf, dtype=jnp.float32)
    accr = jnp.full((_BM,), jnp.inf, dtype=jnp.float32)
    acci = jnp.zeros((_BM,), dtype=jnp.int32)
    lo = 0
    for hi in _BOUNDS:
        bv = jnp.full((_BM,), jnp.inf, dtype=jnp.float32)
        bi = jnp.zeros((_BM,), dtype=jnp.int32)
        start = lo
        while start < hi:
            w = min(_BK, hi - start)
            echunk = emb_ref[start:start + w, :]
            mm = lax.dot_general(zblk.astype(jnp.bfloat16),
                                 echunk.astype(jnp.bfloat16),
                                 (((1,), (1,)), ((), ())),
                                 preferred_element_type=jnp.float32)
            d = (zsq[:, None] + esq_ref[start:start + w][None, :]) - 2.0 * mm
            lmin = jnp.min(d, axis=1)
            larg = jnp.argmin(d, axis=1).astype(jnp.int32) + start
            pred = lmin < bv
            bv = jnp.where(pred, lmin, bv)
            bi = jnp.where(pred, larg, bi)
            start += w
        take = bv < accv
        accv = jnp.where(take, bv.astype(jnp.bfloat16).astype(jnp.float32), accv)
        accr = jnp.where(take, bv, accr)
        acci = jnp.where(take, bi, acci)
        lo = hi
    idx_ref[0, 0, :] = acci
    acc_ref[0, 0] += jnp.sum(accr)


def _argmin_call(zf, emb):
    n, c = zf.shape
    nblocks = n // _BM
    zf3 = zf.reshape(nblocks, _BM, c)
    return pl.pallas_call(
        _argmin_body,
        grid=(nblocks,),
        in_specs=[
            pl.BlockSpec((1, _BM, c), lambda i: (i, 0, 0)),
            pl.BlockSpec(emb.shape, lambda i: (0, 0)),
        ],
        out_specs=[
            pl.BlockSpec((1, 1, _BM), lambda i: (i, 0, 0)),
            pl.BlockSpec(memory_space=pltpu.SMEM),
        ],
        out_shape=[
            jax.ShapeDtypeStruct((nblocks, 1, _BM), jnp.int32),
            jax.ShapeDtypeStruct((1, 1), jnp.float32),
        ],
        scratch_shapes=[pltpu.VMEM((emb.shape[0],), jnp.float32)],
        compiler_params=pltpu.CompilerParams(
            dimension_semantics=("arbitrary",)),
    )(zf3, emb)


def _make_gather(v, d, b):
    info = plsc.get_sparse_core_info()
    nw = info.num_cores * info.num_subcores
    b_per_w = b // nw
    ch = 128
    nch = b_per_w // ch
    mesh = plsc.VectorSubcoreMesh(core_axis_name="c", subcore_axis_name="s")

    @functools.partial(
        pl.kernel, mesh=mesh,
        out_type=jax.ShapeDtypeStruct((b, d), jnp.float32),
        scratch_types=[
            pltpu.VMEM((ch,), jnp.int32),
            pltpu.VMEM((ch, d), jnp.float32),
            pltpu.SemaphoreType.DMA,
        ],
    )
    def k(table_hbm, idx_hbm, out_hbm, idx_v, rows_v, sem):
        wid = lax.axis_index("s") * info.num_cores + lax.axis_index("c")
        base = wid * b_per_w

        def body(cc, carry):
            off = base + cc * ch
            pltpu.sync_copy(idx_hbm.at[pl.ds(off, ch)], idx_v)
            pltpu.async_copy(table_hbm.at[idx_v], rows_v, sem).wait()
            pltpu.sync_copy(rows_v, out_hbm.at[pl.ds(off, ch)])
            return carry

        lax.fori_loop(0, nch, body, 0)

    return k


def kernel(z, embedding):
    b, c, h, w = z.shape
    kk, _ = embedding.shape
    n = b * h * w

    zp = jnp.transpose(z, (0, 2, 3, 1))
    zf = zp.reshape(n, c)

    idx3, acc = _argmin_call(zf, embedding)
    idx = idx3.reshape(n)

    zq = _make_gather(kk, c, n)(embedding, idx)

    loss = (1.0 + _BETA) * acc[0, 0] / (n * c)
    out = jnp.transpose(zq.reshape(b, h, w, c), (0, 3, 1, 2))
    return (out, idx, loss)


# speedup vs baseline: 2.2482x; 2.2482x over previous
import functools

import jax
import jax.numpy as jnp
from jax import lax
from jax.experimental import pallas as pl
from jax.experimental.pallas import tpu as pltpu
from jax.experimental.pallas import tpu_sc as plsc

_BETA = 0.25
_BM = 256
_BK = 512
_BOUNDS = (2736, 5472, 8192)


def _argmin_body(zf_ref, emb_ref, idx_ref, acc_ref, esq_ref):
    i = pl.program_id(0)
    kk = emb_ref.shape[0]

    @pl.when(i == 0)
    def _():
        esq_ref[:] = jnp.sum(emb_ref[:] ** 2, axis=1)
        acc_ref[0, 0] = 0.0

    zblk = zf_ref[0]
    zsq = jnp.sum(zblk ** 2, axis=1)

    jloc = lax.broadcasted_iota(jnp.int32, (_BM, _BK), 1)

    def block_scan(c_lo, c_hi, lo, hi):
        def step(c, carry):
            bv, bi = carry
            base = c * _BK
            echunk = emb_ref[pl.ds(base, _BK), :]
            mm = lax.dot_general(zblk.astype(jnp.bfloat16),
                                 echunk.astype(jnp.bfloat16),
                                 (((1,), (1,)), ((), ())),
                                 preferred_element_type=jnp.float32)
            d = (zsq[:, None] + esq_ref[pl.ds(base, _BK)][None, :]) - 2.0 * mm
            col = base + jloc
            dm = jnp.where((col >= lo) & (col < hi), d, jnp.inf)
            lmin = jnp.min(dm, axis=1)
            larg = jnp.argmin(dm, axis=1).astype(jnp.int32) + base
            pred = lmin < bv
            return jnp.where(pred, lmin, bv), jnp.where(pred, larg, bi)

        bv0 = jnp.full((_BM,), jnp.inf, dtype=jnp.float32)
        bi0 = jnp.zeros((_BM,), dtype=jnp.int32)
        return lax.fori_loop(c_lo, c_hi, step, (bv0, bi0))

    accv = jnp.full((_BM,), jnp.inf, dtype=jnp.float32)
    accr = jnp.full((_BM,), jnp.inf, dtype=jnp.float32)
    acci = jnp.zeros((_BM,), dtype=jnp.int32)
    lo = 0
    for hi in _BOUNDS:
        c_lo, c_hi = lo // _BK, -(-hi // _BK)
        bv, bi = block_scan(c_lo, c_hi, lo, hi)
        take = bv < accv
        accv = jnp.where(take, bv.astype(jnp.bfloat16).astype(jnp.float32), accv)
        accr = jnp.where(take, bv, accr)
        acci = jnp.where(take, bi, acci)
        lo = hi
    idx_ref[0, 0, :] = acci
    acc_ref[0, 0] += jnp.sum(accr)


def _argmin_call(zf, emb):
    n, c = zf.shape
    nblocks = n // _BM
    zf3 = zf.reshape(nblocks, _BM, c)
    return pl.pallas_call(
        _argmin_body,
        grid=(nblocks,),
        in_specs=[
            pl.BlockSpec((1, _BM, c), lambda i: (i, 0, 0)),
            pl.BlockSpec(emb.shape, lambda i: (0, 0)),
        ],
        out_specs=[
            pl.BlockSpec((1, 1, _BM), lambda i: (i, 0, 0)),
            pl.BlockSpec(memory_space=pltpu.SMEM),
        ],
        out_shape=[
            jax.ShapeDtypeStruct((nblocks, 1, _BM), jnp.int32),
            jax.ShapeDtypeStruct((1, 1), jnp.float32),
        ],
        scratch_shapes=[pltpu.VMEM((emb.shape[0],), jnp.float32)],
        compiler_params=pltpu.CompilerParams(
            dimension_semantics=("arbitrary",)),
    )(zf3, emb)


def _make_gather(v, d, b):
    info = plsc.get_sparse_core_info()
    nw = info.num_cores * info.num_subcores
    b_per_w = b // nw
    ch = 128
    nch = b_per_w // ch
    mesh = plsc.VectorSubcoreMesh(core_axis_name="c", subcore_axis_name="s")

    @functools.partial(
        pl.kernel, mesh=mesh,
        out_type=jax.ShapeDtypeStruct((b, d), jnp.float32),
        scratch_types=[
            pltpu.VMEM((ch,), jnp.int32),
            pltpu.VMEM((ch, d), jnp.float32),
            pltpu.SemaphoreType.DMA,
        ],
    )
    def k(table_hbm, idx_hbm, out_hbm, idx_v, rows_v, sem):
        wid = lax.axis_index("s") * info.num_cores + lax.axis_index("c")
        base = wid * b_per_w

        def body(cc, carry):
            off = base + cc * ch
            pltpu.sync_copy(idx_hbm.at[pl.ds(off, ch)], idx_v)
            pltpu.async_copy(table_hbm.at[idx_v], rows_v, sem).wait()
            pltpu.sync_copy(rows_v, out_hbm.at[pl.ds(off, ch)])
            return carry

        lax.fori_loop(0, nch, body, 0)

    return k


def kernel(z, embedding):
    b, c, h, w = z.shape
    kk, _ = embedding.shape
    n = b * h * w

    zp = jnp.transpose(z, (0, 2, 3, 1))
    zf = zp.reshape(n, c)

    idx3, acc = _argmin_call(zf, embedding)
    idx = idx3.reshape(n)

    zq = _make_gather(kk, c, n)(embedding, idx)

    loss = (1.0 + _BETA) * acc[0, 0] / (n * c)
    out = jnp.transpose(zq.reshape(b, h, w, c), (0, 3, 1, 2))
    return (out, idx, loss)


# speedup vs baseline: 2.4235x; 1.0780x over previous
import functools

import jax
import jax.numpy as jnp
from jax import lax
from jax.experimental import pallas as pl
from jax.experimental.pallas import tpu as pltpu
from jax.experimental.pallas import tpu_sc as plsc

_BETA = 0.25
_BM = 512
_BK = 512
_BOUNDS = (2736, 5472, 8192)


def _argmin_body(zf_ref, emb_ref, idx_ref, acc_ref, esq_ref):
    i = pl.program_id(0)
    kk = emb_ref.shape[0]

    @pl.when(i == 0)
    def _():
        esq_ref[:] = jnp.sum(emb_ref[:] ** 2, axis=1)
        acc_ref[0, 0] = 0.0

    zblk = zf_ref[0]
    zsq = jnp.sum(zblk ** 2, axis=1)

    jloc = lax.broadcasted_iota(jnp.int32, (_BM, _BK), 1)

    def chunk_d(base):
        echunk = emb_ref[pl.ds(base, _BK), :]
        mm = lax.dot_general(zblk.astype(jnp.bfloat16),
                             echunk.astype(jnp.bfloat16),
                             (((1,), (1,)), ((), ())),
                             preferred_element_type=jnp.float32)
        return (zsq[:, None] + esq_ref[pl.ds(base, _BK)][None, :]) - 2.0 * mm

    def combine(carry, lmin, larg):
        bv, bi = carry
        pred = lmin < bv
        return jnp.where(pred, lmin, bv), jnp.where(pred, larg, bi)

    def interior(c, carry):
        d = chunk_d(c * _BK)
        return combine(carry, jnp.min(d, axis=1),
                       jnp.argmin(d, axis=1).astype(jnp.int32) + c * _BK)

    def edge(base, lo, hi, carry):
        d = chunk_d(base)
        col = base + jloc
        dm = jnp.where((col >= lo) & (col < hi), d, jnp.inf)
        return combine(carry, jnp.min(dm, axis=1),
                       jnp.argmin(dm, axis=1).astype(jnp.int32) + base)

    def fresh():
        return (jnp.full((_BM,), jnp.inf, dtype=jnp.float32),
                jnp.zeros((_BM,), dtype=jnp.int32))

    c0, c1 = _BOUNDS[0] // _BK, _BOUNDS[1] // _BK
    b0 = lax.fori_loop(0, c0, interior, fresh())
    b0 = edge(c0 * _BK, 0, _BOUNDS[0], b0)
    b1 = edge(c0 * _BK, _BOUNDS[0], (c0 + 1) * _BK, fresh())
    b1 = lax.fori_loop(c0 + 1, c1, interior, b1)
    b1 = edge(c1 * _BK, (c1 * _BK), _BOUNDS[1], b1)
    b2 = edge(c1 * _BK, _BOUNDS[1], (c1 + 1) * _BK, fresh())
    b2 = lax.fori_loop(c1 + 1, kk // _BK, interior, b2)

    accv = jnp.full((_BM,), jnp.inf, dtype=jnp.float32)
    accr = jnp.full((_BM,), jnp.inf, dtype=jnp.float32)
    acci = jnp.zeros((_BM,), dtype=jnp.int32)
    for bv, bi in (b0, b1, b2):
        take = bv < accv
        accv = jnp.where(take, bv.astype(jnp.bfloat16).astype(jnp.float32), accv)
        accr = jnp.where(take, bv, accr)
        acci = jnp.where(take, bi, acci)
    idx_ref[0, 0, :] = acci
    acc_ref[0, 0] += jnp.sum(accr)


def _argmin_call(zf, emb):
    n, c = zf.shape
    nblocks = n // _BM
    zf3 = zf.reshape(nblocks, _BM, c)
    return pl.pallas_call(
        _argmin_body,
        grid=(nblocks,),
        in_specs=[
            pl.BlockSpec((1, _BM, c), lambda i: (i, 0, 0)),
            pl.BlockSpec(emb.shape, lambda i: (0, 0)),
        ],
        out_specs=[
            pl.BlockSpec((1, 1, _BM), lambda i: (i, 0, 0)),
            pl.BlockSpec(memory_space=pltpu.SMEM),
        ],
        out_shape=[
            jax.ShapeDtypeStruct((nblocks, 1, _BM), jnp.int32),
            jax.ShapeDtypeStruct((1, 1), jnp.float32),
        ],
        scratch_shapes=[pltpu.VMEM((emb.shape[0],), jnp.float32)],
        compiler_params=pltpu.CompilerParams(
            dimension_semantics=("arbitrary",)),
    )(zf3, emb)


def _make_gather(v, d, b):
    info = plsc.get_sparse_core_info()
    nw = info.num_cores * info.num_subcores
    b_per_w = b // nw
    ch = 128
    nch = b_per_w // ch
    mesh = plsc.VectorSubcoreMesh(core_axis_name="c", subcore_axis_name="s")

    @functools.partial(
        pl.kernel, mesh=mesh,
        out_type=jax.ShapeDtypeStruct((b, d), jnp.float32),
        scratch_types=[
            pltpu.VMEM((ch,), jnp.int32),
            pltpu.VMEM((ch, d), jnp.float32),
            pltpu.SemaphoreType.DMA,
        ],
    )
    def k(table_hbm, idx_hbm, out_hbm, idx_v, rows_v, sem):
        wid = lax.axis_index("s") * info.num_cores + lax.axis_index("c")
        base = wid * b_per_w

        def body(cc, carry):
            off = base + cc * ch
            pltpu.sync_copy(idx_hbm.at[pl.ds(off, ch)], idx_v)
            pltpu.async_copy(table_hbm.at[idx_v], rows_v, sem).wait()
            pltpu.sync_copy(rows_v, out_hbm.at[pl.ds(off, ch)])
            return carry

        lax.fori_loop(0, nch, body, 0)

    return k


def kernel(z, embedding):
    b, c, h, w = z.shape
    kk, _ = embedding.shape
    n = b * h * w

    zp = jnp.transpose(z, (0, 2, 3, 1))
    zf = zp.reshape(n, c)

    idx3, acc = _argmin_call(zf, embedding)
    idx = idx3.reshape(n)

    zq = _make_gather(kk, c, n)(embedding, idx)

    loss = (1.0 + _BETA) * acc[0, 0] / (n * c)
    out = jnp.transpose(zq.reshape(b, h, w, c), (0, 3, 1, 2))
    return (out, idx, loss)


# speedup vs baseline: 3.6059x; 1.4879x over previous
import functools

import jax
import jax.numpy as jnp
from jax import lax
from jax.experimental import pallas as pl
from jax.experimental.pallas import tpu as pltpu
from jax.experimental.pallas import tpu_sc as plsc

_BETA = 0.25
_BM = 512
_BK = 1024
_BOUNDS = (2736, 5472, 8192)


def _argmin_body(zf_ref, emb_ref, idx_ref, acc_ref, esq_ref, nemb_ref):
    i = pl.program_id(0)
    kk = emb_ref.shape[0]

    @pl.when(i == 0)
    def _():
        esq_ref[:] = jnp.sum(emb_ref[:] ** 2, axis=1)
        nemb_ref[:] = (jnp.float32(-2.0) * emb_ref[:]).astype(jnp.bfloat16)
        acc_ref[0, 0] = 0.0

    zblk = zf_ref[0]
    zsq = jnp.sum(zblk ** 2, axis=1)
    zbf = zblk.astype(jnp.bfloat16)

    jloc = lax.broadcasted_iota(jnp.int32, (_BM, _BK), 1)

    def chunk_d(base):
        echunk = nemb_ref[pl.ds(base, _BK), :]
        mm2 = lax.dot_general(zbf, echunk,
                              (((1,), (1,)), ((), ())),
                              preferred_element_type=jnp.float32)
        return (zsq[:, None] + esq_ref[pl.ds(base, _BK)][None, :]) + mm2

    def combine(carry, lmin, larg):
        bv, bi = carry
        pred = lmin < bv
        return jnp.where(pred, lmin, bv), jnp.where(pred, larg, bi)

    def interior(c, carry):
        d = chunk_d(c * _BK)
        return combine(carry, jnp.min(d, axis=1),
                       jnp.argmin(d, axis=1).astype(jnp.int32) + c * _BK)

    def edge(base, lo, hi, carry):
        d = chunk_d(base)
        col = base + jloc
        dm = jnp.where((col >= lo) & (col < hi), d, jnp.inf)
        return combine(carry, jnp.min(dm, axis=1),
                       jnp.argmin(dm, axis=1).astype(jnp.int32) + base)

    def fresh():
        return (jnp.full((_BM,), jnp.inf, dtype=jnp.float32),
                jnp.zeros((_BM,), dtype=jnp.int32))

    c0, c1 = _BOUNDS[0] // _BK, _BOUNDS[1] // _BK
    b0 = lax.fori_loop(0, c0, interior, fresh())
    b0 = edge(c0 * _BK, 0, _BOUNDS[0], b0)
    b1 = edge(c0 * _BK, _BOUNDS[0], (c0 + 1) * _BK, fresh())
    b1 = lax.fori_loop(c0 + 1, c1, interior, b1)
    b1 = edge(c1 * _BK, (c1 * _BK), _BOUNDS[1], b1)
    b2 = edge(c1 * _BK, _BOUNDS[1], (c1 + 1) * _BK, fresh())
    b2 = lax.fori_loop(c1 + 1, kk // _BK, interior, b2)

    accv = jnp.full((_BM,), jnp.inf, dtype=jnp.float32)
    accr = jnp.full((_BM,), jnp.inf, dtype=jnp.float32)
    acci = jnp.zeros((_BM,), dtype=jnp.int32)
    for bv, bi in (b0, b1, b2):
        take = bv < accv
        accv = jnp.where(take, bv.astype(jnp.bfloat16).astype(jnp.float32), accv)
        accr = jnp.where(take, bv, accr)
        acci = jnp.where(take, bi, acci)
    idx_ref[0, 0, :] = acci
    acc_ref[0, 0] += jnp.sum(accr)


def _argmin_call(zf, emb):
    n, c = zf.shape
    nblocks = n // _BM
    zf3 = zf.reshape(nblocks, _BM, c)
    return pl.pallas_call(
        _argmin_body,
        grid=(nblocks,),
        in_specs=[
            pl.BlockSpec((1, _BM, c), lambda i: (i, 0, 0)),
            pl.BlockSpec(emb.shape, lambda i: (0, 0)),
        ],
        out_specs=[
            pl.BlockSpec((1, 1, _BM), lambda i: (i, 0, 0)),
            pl.BlockSpec(memory_space=pltpu.SMEM),
        ],
        out_shape=[
            jax.ShapeDtypeStruct((nblocks, 1, _BM), jnp.int32),
            jax.ShapeDtypeStruct((1, 1), jnp.float32),
        ],
        scratch_shapes=[pltpu.VMEM((emb.shape[0],), jnp.float32),
                        pltpu.VMEM(emb.shape, jnp.bfloat16)],
        compiler_params=pltpu.CompilerParams(
            dimension_semantics=("arbitrary",)),
    )(zf3, emb)


def _make_gather(v, d, b):
    info = plsc.get_sparse_core_info()
    nw = info.num_cores * info.num_subcores
    b_per_w = b // nw
    ch = 128
    nch = b_per_w // ch
    mesh = plsc.VectorSubcoreMesh(core_axis_name="c", subcore_axis_name="s")

    @functools.partial(
        pl.kernel, mesh=mesh,
        out_type=jax.ShapeDtypeStruct((b, d), jnp.float32),
        scratch_types=[
            pltpu.VMEM((ch,), jnp.int32),
            pltpu.VMEM((ch, d), jnp.float32),
            pltpu.SemaphoreType.DMA,
        ],
    )
    def k(table_hbm, idx_hbm, out_hbm, idx_v, rows_v, sem):
        wid = lax.axis_index("s") * info.num_cores + lax.axis_index("c")
        base = wid * b_per_w

        def body(cc, carry):
            off = base + cc * ch
            pltpu.sync_copy(idx_hbm.at[pl.ds(off, ch)], idx_v)
            pltpu.async_copy(table_hbm.at[idx_v], rows_v, sem).wait()
            pltpu.sync_copy(rows_v, out_hbm.at[pl.ds(off, ch)])
            return carry

        lax.fori_loop(0, nch, body, 0)

    return k


def kernel(z, embedding):
    b, c, h, w = z.shape
    kk, _ = embedding.shape
    n = b * h * w

    zp = jnp.transpose(z, (0, 2, 3, 1))
    zf = zp.reshape(n, c)

    idx3, acc = _argmin_call(zf, embedding)
    idx = idx3.reshape(n)

    zq = _make_gather(kk, c, n)(embedding, idx)

    loss = (1.0 + _BETA) * acc[0, 0] / (n * c)
    out = jnp.transpose(zq.reshape(b, h, w, c), (0, 3, 1, 2))
    return (out, idx, loss)
